# baseline (speedup 1.0000x reference)
import functools

import jax
import jax.numpy as jnp
from jax import lax
from jax.experimental import pallas as pl
from jax.experimental.pallas import tpu as pltpu
from jax.experimental.pallas import tpu_sc as plsc

NC = 2
NS = 16
CHUNK = 128
NBUF = 2
NHALF = 2
RADIUS = 3


def _sc_segment_sum(h, srcp2, dstp2, zinit, n, npad, chunks_per_worker):
    d = h.shape[1]
    t_w = chunks_per_worker
    t_h = t_w // NHALF
    rows_per_tile = npad // NS
    mesh = plsc.VectorSubcoreMesh(core_axis_name="c", subcore_axis_name="s")

    @functools.partial(
        pl.kernel,
        out_type=jax.ShapeDtypeStruct((NC, npad, d), jnp.float32),
        mesh=mesh,
        scratch_types=[
            pltpu.VMEM((t_h, CHUNK), jnp.int32),
            pltpu.VMEM((t_h, CHUNK), jnp.int32),
            pltpu.VMEM((NBUF, CHUNK, d), jnp.float32),
            pltpu.VMEM_SHARED((npad, d), jnp.float32),
            pltpu.SemaphoreType.DMA((NBUF,)),
        ],
    )
    def seg_kernel(h_hbm, src_hbm, dst_hbm, z_hbm, out_hbm, sidx, didx, rows, acc, sems):
        c = lax.axis_index("c")
        s = lax.axis_index("s")
        wid = c * NS + s
        cbase = wid * t_w
        pltpu.sync_copy(z_hbm.at[pl.ds(s * rows_per_tile, rows_per_tile)],
                        acc.at[pl.ds(s * rows_per_tile, rows_per_tile)])
        plsc.subcore_barrier()

        def gather_start(t, b):
            pltpu.async_copy(h_hbm.at[sidx.at[t]], rows.at[b], sems.at[b])

        def gather_wait(t, b):
            pltpu.make_async_copy(h_hbm.at[sidx.at[t]], rows.at[b], sems.at[b]).wait()

        for half in range(NHALF):
            pltpu.sync_copy(src_hbm.at[pl.ds(cbase + half * t_h, t_h)], sidx)
            pltpu.sync_copy(dst_hbm.at[pl.ds(cbase + half * t_h, t_h)], didx)
            for b in range(NBUF):
                gather_start(b, b)

            def outer(i, carry):
                t0 = i * NBUF
                for b in range(NBUF):
                    t = t0 + b
                    gather_wait(t, b)
                    pltpu.sync_copy(rows.at[b], acc.at[didx.at[t]], add=True)

                    @pl.when(t + NBUF < t_h)
                    def _next():
                        gather_start(t + NBUF, b)
                return carry

            lax.fori_loop(0, t_h // NBUF, outer, 0, unroll=False)
        plsc.subcore_barrier()
        pltpu.sync_copy(acc.at[pl.ds(s * rows_per_tile, rows_per_tile)],
                        out_hbm.at[c, pl.ds(s * rows_per_tile, rows_per_tile)])

    return seg_kernel(h, srcp2, dstp2, zinit)


def _tc_step(h, parts, W1, b1r, W2, b2r, fp_in, blk):
    n, d = h.shape
    f = W2.shape[1]
    grid = (n // blk,)

    def body(h_ref, p_ref, w1_ref, b1_ref, w2_ref, b2_ref, fpin_ref,
             hn_ref, fp_ref):
        upd = h_ref[...] + p_ref[0] + p_ref[1]
        hsh = jnp.maximum(
            jnp.dot(upd, w1_ref[...], preferred_element_type=jnp.float32)
            + b1_ref[...], 0.0)
        hn_ref[...] = hsh
        logits = (jnp.dot(hsh.astype(jnp.bfloat16),
                          w2_ref[...].astype(jnp.bfloat16),
                          preferred_element_type=jnp.float32)
                  + b2_ref[...])
        m = jnp.max(logits, axis=1, keepdims=True)
        e = jnp.exp(logits - m)
        ssum = jnp.sum(e, axis=1, keepdims=True)
        contrib = jnp.sum(e / ssum, axis=0, keepdims=True)

        @pl.when(pl.program_id(0) == 0)
        def _init():
            fp_ref[...] = fpin_ref[...]

        fp_ref[...] += contrib

    return pl.pallas_call(
        body,
        grid=grid,
        in_specs=[
            pl.BlockSpec((blk, d), lambda i: (i, 0)),
            pl.BlockSpec((NC, blk, d), lambda i: (0, i, 0)),
            pl.BlockSpec((d, d), lambda i: (0, 0)),
            pl.BlockSpec((1, d), lambda i: (0, 0)),
            pl.BlockSpec((d, f), lambda i: (0, 0)),
            pl.BlockSpec((1, f), lambda i: (0, 0)),
            pl.BlockSpec((1, f), lambda i: (0, 0)),
        ],
        out_specs=[
            pl.BlockSpec((blk, d), lambda i: (i, 0)),
            pl.BlockSpec((1, f), lambda i: (0, 0)),
        ],
        out_shape=[
            jax.ShapeDtypeStruct((n, d), jnp.float32),
            jax.ShapeDtypeStruct((1, f), jnp.float32),
        ],
    )(h, parts, W1, b1r, W2, b2r, fp_in)


def kernel(x, edge_index, W1, b1, W2, b2):
    n, d = x.shape
    f = W2.shape[1]
    src = edge_index[0].astype(jnp.int32)
    dst = edge_index[1].astype(jnp.int32)
    e = src.shape[0]

    nw = NC * NS
    per_round = nw * CHUNK * NBUF * NHALF
    ep = ((e + per_round - 1) // per_round) * per_round
    chunks_per_worker = ep // (nw * CHUNK)
    npad = ((n + 1 + NS * 8 - 1) // (NS * 8)) * (NS * 8)
    pad_ar = jnp.arange(ep - e, dtype=jnp.int32)
    pad_src = pad_ar % n
    pad_dst = n + pad_ar % (npad - n)

    assert n <= (1 << 14)
    packed = jnp.concatenate([(src << 14) + dst, (pad_src << 14) + pad_dst])
    packed = packed.reshape(nw, ep // nw)
    span = n << 14
    offs = ((jnp.arange(nw, dtype=jnp.int32) * (n // nw)) << 14)[:, None]
    packed = (jnp.sort((packed - offs) % span, axis=1) + offs) % span
    srcp_flat = packed >> 14
    dstp_flat = packed & ((1 << 14) - 1)

    def _layout(a):
        a = a.reshape(nw, chunks_per_worker, CHUNK)
        a = jnp.swapaxes(a, 1, 2)
        return a.reshape(-1, CHUNK)

    srcp = _layout(srcp_flat)
    dstp = _layout(dstp_flat)
    zinit = jnp.zeros((npad, d), jnp.float32)

    b1r = b1.reshape(1, d)
    b2r = b2.reshape(1, f)
    fp = jnp.zeros((1, f), jnp.float32)
    h = x
    for _ in range(RADIUS):
        parts = _sc_segment_sum(h, srcp, dstp, zinit, n, npad, chunks_per_worker)
        h, fp = _tc_step(h, parts, W1, b1r, W2, b2r, fp, blk=1000)
    return fp


# speedup vs baseline: 1.3677x; 1.3677x over previous
import functools

import jax
import jax.numpy as jnp
from jax import lax
from jax.experimental import pallas as pl
from jax.experimental.pallas import tpu as pltpu
from jax.experimental.pallas import tpu_sc as plsc

NC = 2
NS = 16
CHUNK = 128
NBUF = 2
NHALF = 2
RADIUS = 3


def _sc_segment_sum(h, srcp2, dstp2, zinit, n, npad, chunks_per_worker):
    d = h.shape[1]
    t_w = chunks_per_worker
    t_h = t_w // NHALF
    rows_per_tile = npad // NS
    mesh = plsc.VectorSubcoreMesh(core_axis_name="c", subcore_axis_name="s")

    @functools.partial(
        pl.kernel,
        out_type=jax.ShapeDtypeStruct((NC, npad, d), jnp.float32),
        mesh=mesh,
        scratch_types=[
            pltpu.VMEM((t_h, CHUNK), jnp.int32),
            pltpu.VMEM((t_h, CHUNK), jnp.int32),
            pltpu.VMEM((NBUF, CHUNK, d), jnp.float32),
            pltpu.VMEM_SHARED((npad, d), jnp.float32),
            pltpu.SemaphoreType.DMA((NBUF,)),
        ],
    )
    def seg_kernel(h_hbm, src_hbm, dst_hbm, z_hbm, out_hbm, sidx, didx, rows, acc, sems):
        c = lax.axis_index("c")
        s = lax.axis_index("s")
        wid = c * NS + s
        cbase = wid * t_w
        pltpu.sync_copy(z_hbm.at[pl.ds(s * rows_per_tile, rows_per_tile)],
                        acc.at[pl.ds(s * rows_per_tile, rows_per_tile)])
        plsc.subcore_barrier()

        def gather_start(t, b):
            pltpu.async_copy(h_hbm.at[sidx.at[t]], rows.at[b], sems.at[b])

        def gather_wait(t, b):
            pltpu.make_async_copy(h_hbm.at[sidx.at[t]], rows.at[b], sems.at[b]).wait()

        for half in range(NHALF):
            pltpu.sync_copy(src_hbm.at[pl.ds(cbase + half * t_h, t_h)], sidx)
            pltpu.sync_copy(dst_hbm.at[pl.ds(cbase + half * t_h, t_h)], didx)
            for b in range(NBUF):
                gather_start(b, b)

            def outer(i, carry):
                t0 = i * NBUF
                for b in range(NBUF):
                    t = t0 + b
                    gather_wait(t, b)
                    pltpu.sync_copy(rows.at[b], acc.at[didx.at[t]], add=True)

                    @pl.when(t + NBUF < t_h)
                    def _next():
                        gather_start(t + NBUF, b)
                return carry

            lax.fori_loop(0, t_h // NBUF, outer, 0, unroll=False)
        plsc.subcore_barrier()
        pltpu.sync_copy(acc.at[pl.ds(s * rows_per_tile, rows_per_tile)],
                        out_hbm.at[c, pl.ds(s * rows_per_tile, rows_per_tile)])

    return seg_kernel(h, srcp2, dstp2, zinit)


def _tc_step(h, parts, W1, b1r, W2, b2r, fp_in, blk):
    n, d = h.shape
    f = W2.shape[1]
    grid = (n // blk,)

    def body(h_ref, p_ref, w1_ref, b1_ref, w2_ref, b2_ref, fpin_ref,
             hn_ref, fp_ref):
        upd = h_ref[...] + p_ref[0] + p_ref[1]
        hsh = jnp.maximum(
            jnp.dot(upd, w1_ref[...], preferred_element_type=jnp.float32)
            + b1_ref[...], 0.0)
        hn_ref[...] = hsh
        logits = (jnp.dot(hsh.astype(jnp.bfloat16),
                          w2_ref[...].astype(jnp.bfloat16),
                          preferred_element_type=jnp.float32)
                  + b2_ref[...])
        m = jnp.max(logits, axis=1, keepdims=True)
        e = jnp.exp(logits - m)
        ssum = jnp.sum(e, axis=1, keepdims=True)
        contrib = jnp.sum(e / ssum, axis=0, keepdims=True)

        @pl.when(pl.program_id(0) == 0)
        def _init():
            fp_ref[...] = fpin_ref[...]

        fp_ref[...] += contrib

    return pl.pallas_call(
        body,
        grid=grid,
        in_specs=[
            pl.BlockSpec((blk, d), lambda i: (i, 0)),
            pl.BlockSpec((NC, blk, d), lambda i: (0, i, 0)),
            pl.BlockSpec((d, d), lambda i: (0, 0)),
            pl.BlockSpec((1, d), lambda i: (0, 0)),
            pl.BlockSpec((d, f), lambda i: (0, 0)),
            pl.BlockSpec((1, f), lambda i: (0, 0)),
            pl.BlockSpec((1, f), lambda i: (0, 0)),
        ],
        out_specs=[
            pl.BlockSpec((blk, d), lambda i: (i, 0)),
            pl.BlockSpec((1, f), lambda i: (0, 0)),
        ],
        out_shape=[
            jax.ShapeDtypeStruct((n, d), jnp.float32),
            jax.ShapeDtypeStruct((1, f), jnp.float32),
        ],
    )(h, parts, W1, b1r, W2, b2r, fp_in)


def kernel(x, edge_index, W1, b1, W2, b2):
    n, d = x.shape
    f = W2.shape[1]
    src = edge_index[0].astype(jnp.int32)
    dst = edge_index[1].astype(jnp.int32)
    e = src.shape[0]

    nw = NC * NS
    per_round = nw * CHUNK * NBUF * NHALF
    ep = ((e + per_round - 1) // per_round) * per_round
    chunks_per_worker = ep // (nw * CHUNK)
    npad = ((n + 1 + NS * 8 - 1) // (NS * 8)) * (NS * 8)
    pad_ar = jnp.arange(ep - e, dtype=jnp.int32)
    pad_src = pad_ar % n
    pad_dst = n + pad_ar % (npad - n)

    assert n <= (1 << 14)
    subg = 8
    ngrp = nw * subg
    packed = jnp.concatenate([(src << 14) + dst, (pad_src << 14) + pad_dst])
    packed = packed.reshape(ngrp, ep // ngrp)
    span = n << 14
    offs = ((jnp.arange(ngrp, dtype=jnp.int32) * (n // ngrp)) << 14)[:, None]
    packed = (jnp.sort((packed - offs) % span, axis=1) + offs) % span
    srcp_flat = packed >> 14
    dstp_flat = packed & ((1 << 14) - 1)

    def _layout(a):
        a = a.reshape(nw, chunks_per_worker, CHUNK)
        a = jnp.swapaxes(a, 1, 2)
        return a.reshape(-1, CHUNK)

    srcp = _layout(srcp_flat)
    dstp = _layout(dstp_flat)
    zinit = jnp.zeros((npad, d), jnp.float32)

    b1r = b1.reshape(1, d)
    b2r = b2.reshape(1, f)
    fp = jnp.zeros((1, f), jnp.float32)
    h = x
    for _ in range(RADIUS):
        parts = _sc_segment_sum(h, srcp, dstp, zinit, n, npad, chunks_per_worker)
        h, fp = _tc_step(h, parts, W1, b1r, W2, b2r, fp, blk=1000)
    return fp


# speedup vs baseline: 1.4006x; 1.0241x over previous
import functools

import jax
import jax.numpy as jnp
from jax import lax
from jax.experimental import pallas as pl
from jax.experimental.pallas import tpu as pltpu
from jax.experimental.pallas import tpu_sc as plsc

NC = 2
NS = 16
CHUNK = 128
NBUF = 2
NHALF = 2
RADIUS = 3


def _sc_segment_sum(h, srcp2, dstp2, zinit, n, npad, chunks_per_worker):
    d = h.shape[1]
    t_w = chunks_per_worker
    t_h = t_w // NHALF
    rows_per_tile = npad // NS
    mesh = plsc.VectorSubcoreMesh(core_axis_name="c", subcore_axis_name="s")

    @functools.partial(
        pl.kernel,
        out_type=jax.ShapeDtypeStruct((NC, npad, d), jnp.float32),
        mesh=mesh,
        scratch_types=[
            pltpu.VMEM((t_h, CHUNK), jnp.int32),
            pltpu.VMEM((t_h, CHUNK), jnp.int32),
            pltpu.VMEM((NBUF, CHUNK, d), jnp.float32),
            pltpu.VMEM_SHARED((npad, d), jnp.float32),
            pltpu.SemaphoreType.DMA((NBUF,)),
        ],
    )
    def seg_kernel(h_hbm, src_hbm, dst_hbm, z_hbm, out_hbm, sidx, didx, rows, acc, sems):
        c = lax.axis_index("c")
        s = lax.axis_index("s")
        wid = c * NS + s
        cbase = wid * t_w
        pltpu.sync_copy(z_hbm.at[pl.ds(s * rows_per_tile, rows_per_tile)],
                        acc.at[pl.ds(s * rows_per_tile, rows_per_tile)])
        plsc.subcore_barrier()

        def gather_start(t, b):
            pltpu.async_copy(h_hbm.at[sidx.at[t]], rows.at[b], sems.at[b])

        def gather_wait(t, b):
            pltpu.make_async_copy(h_hbm.at[sidx.at[t]], rows.at[b], sems.at[b]).wait()

        for half in range(NHALF):
            pltpu.sync_copy(src_hbm.at[pl.ds(cbase + half * t_h, t_h)], sidx)
            pltpu.sync_copy(dst_hbm.at[pl.ds(cbase + half * t_h, t_h)], didx)
            for b in range(NBUF):
                gather_start(b, b)

            def outer(i, carry):
                t0 = i * NBUF
                for b in range(NBUF):
                    t = t0 + b
                    gather_wait(t, b)
                    pltpu.sync_copy(rows.at[b], acc.at[didx.at[t]], add=True)

                    @pl.when(t + NBUF < t_h)
                    def _next():
                        gather_start(t + NBUF, b)
                return carry

            lax.fori_loop(0, t_h // NBUF, outer, 0, unroll=False)
        plsc.subcore_barrier()
        pltpu.sync_copy(acc.at[pl.ds(s * rows_per_tile, rows_per_tile)],
                        out_hbm.at[c, pl.ds(s * rows_per_tile, rows_per_tile)])

    return seg_kernel(h, srcp2, dstp2, zinit)


def _tc_step(h, parts, W1, b1r, W2, b2r, fp_in, blk):
    n, d = h.shape
    f = W2.shape[1]
    grid = (n // blk,)

    def body(h_ref, p_ref, w1_ref, b1_ref, w2_ref, b2_ref, fpin_ref,
             hn_ref, fp_ref):
        upd = h_ref[...] + p_ref[0] + p_ref[1]
        hsh = jnp.maximum(
            jnp.dot(upd, w1_ref[...], preferred_element_type=jnp.float32)
            + b1_ref[...], 0.0)
        hn_ref[...] = hsh
        logits = (jnp.dot(hsh.astype(jnp.bfloat16),
                          w2_ref[...].astype(jnp.bfloat16),
                          preferred_element_type=jnp.float32)
                  + b2_ref[...])
        m = jnp.max(logits, axis=1, keepdims=True)
        e = jnp.exp(logits - m)
        ssum = jnp.sum(e, axis=1, keepdims=True)
        contrib = jnp.sum(e / ssum, axis=0, keepdims=True)

        @pl.when(pl.program_id(0) == 0)
        def _init():
            fp_ref[...] = fpin_ref[...]

        fp_ref[...] += contrib

    return pl.pallas_call(
        body,
        grid=grid,
        in_specs=[
            pl.BlockSpec((blk, d), lambda i: (i, 0)),
            pl.BlockSpec((NC, blk, d), lambda i: (0, i, 0)),
            pl.BlockSpec((d, d), lambda i: (0, 0)),
            pl.BlockSpec((1, d), lambda i: (0, 0)),
            pl.BlockSpec((d, f), lambda i: (0, 0)),
            pl.BlockSpec((1, f), lambda i: (0, 0)),
            pl.BlockSpec((1, f), lambda i: (0, 0)),
        ],
        out_specs=[
            pl.BlockSpec((blk, d), lambda i: (i, 0)),
            pl.BlockSpec((1, f), lambda i: (0, 0)),
        ],
        out_shape=[
            jax.ShapeDtypeStruct((n, d), jnp.float32),
            jax.ShapeDtypeStruct((1, f), jnp.float32),
        ],
    )(h, parts, W1, b1r, W2, b2r, fp_in)


def kernel(x, edge_index, W1, b1, W2, b2):
    n, d = x.shape
    f = W2.shape[1]
    src = edge_index[0].astype(jnp.int32)
    dst = edge_index[1].astype(jnp.int32)
    e = src.shape[0]

    nw = NC * NS
    per_round = nw * CHUNK * NBUF * NHALF
    ep = ((e + per_round - 1) // per_round) * per_round
    chunks_per_worker = ep // (nw * CHUNK)
    npad = ((n + 1 + NS * 8 - 1) // (NS * 8)) * (NS * 8)
    pad_ar = jnp.arange(ep - e, dtype=jnp.int32)
    pad_src = pad_ar % n
    pad_dst = n + pad_ar % (npad - n)

    assert n <= (1 << 14)
    subg = 32
    ngrp = nw * subg
    packed = jnp.concatenate([(src << 14) + dst, (pad_src << 14) + pad_dst])
    packed = packed.reshape(ngrp, ep // ngrp)
    span = n << 14
    offs = ((jnp.arange(ngrp, dtype=jnp.int32) * (n // ngrp)) << 14)[:, None]
    packed = (jnp.sort((packed - offs) % span, axis=1) + offs) % span
    srcp_flat = packed >> 14
    dstp_flat = packed & ((1 << 14) - 1)

    def _layout(a):
        a = a.reshape(nw, chunks_per_worker, CHUNK)
        a = jnp.swapaxes(a, 1, 2)
        return a.reshape(-1, CHUNK)

    srcp = _layout(srcp_flat)
    dstp = _layout(dstp_flat)
    zinit = jnp.zeros((npad, d), jnp.float32)

    b1r = b1.reshape(1, d)
    b2r = b2.reshape(1, f)
    fp = jnp.zeros((1, f), jnp.float32)
    h = x
    for _ in range(RADIUS):
        parts = _sc_segment_sum(h, srcp, dstp, zinit, n, npad, chunks_per_worker)
        h, fp = _tc_step(h, parts, W1, b1r, W2, b2r, fp, blk=1000)
    return fp


# speedup vs baseline: 1.4617x; 1.0436x over previous
import functools

import jax
import jax.numpy as jnp
from jax import lax
from jax.experimental import pallas as pl
from jax.experimental.pallas import tpu as pltpu
from jax.experimental.pallas import tpu_sc as plsc

NC = 2
NS = 16
CHUNK = 64
NBUF = 4
NHALF = 4
RADIUS = 3


def _sc_segment_sum(h, srcp2, dstp2, zinit, n, npad, chunks_per_worker):
    d = h.shape[1]
    t_w = chunks_per_worker
    t_h = t_w // NHALF
    rows_per_tile = npad // NS
    mesh = plsc.VectorSubcoreMesh(core_axis_name="c", subcore_axis_name="s")

    @functools.partial(
        pl.kernel,
        out_type=jax.ShapeDtypeStruct((NC, npad, d), jnp.float32),
        mesh=mesh,
        scratch_types=[
            pltpu.VMEM((t_h, CHUNK), jnp.int32),
            pltpu.VMEM((t_h, CHUNK), jnp.int32),
            pltpu.VMEM((NBUF, CHUNK, d), jnp.float32),
            pltpu.VMEM_SHARED((npad, d), jnp.float32),
            pltpu.SemaphoreType.DMA((NBUF,)),
        ],
    )
    def seg_kernel(h_hbm, src_hbm, dst_hbm, z_hbm, out_hbm, sidx, didx, rows, acc, sems):
        c = lax.axis_index("c")
        s = lax.axis_index("s")
        wid = c * NS + s
        cbase = wid * t_w
        pltpu.sync_copy(z_hbm.at[pl.ds(s * rows_per_tile, rows_per_tile)],
                        acc.at[pl.ds(s * rows_per_tile, rows_per_tile)])
        plsc.subcore_barrier()

        def gather_start(t, b):
            pltpu.async_copy(h_hbm.at[sidx.at[t]], rows.at[b], sems.at[b])

        def gather_wait(t, b):
            pltpu.make_async_copy(h_hbm.at[sidx.at[t]], rows.at[b], sems.at[b]).wait()

        for half in range(NHALF):
            pltpu.sync_copy(src_hbm.at[pl.ds(cbase + half * t_h, t_h)], sidx)
            pltpu.sync_copy(dst_hbm.at[pl.ds(cbase + half * t_h, t_h)], didx)
            for b in range(NBUF):
                gather_start(b, b)

            def outer(i, carry):
                t0 = i * NBUF
                for b in range(NBUF):
                    t = t0 + b
                    gather_wait(t, b)
                    pltpu.sync_copy(rows.at[b], acc.at[didx.at[t]], add=True)

                    @pl.when(t + NBUF < t_h)
                    def _next():
                        gather_start(t + NBUF, b)
                return carry

            lax.fori_loop(0, t_h // NBUF, outer, 0, unroll=False)
        plsc.subcore_barrier()
        pltpu.sync_copy(acc.at[pl.ds(s * rows_per_tile, rows_per_tile)],
                        out_hbm.at[c, pl.ds(s * rows_per_tile, rows_per_tile)])

    return seg_kernel(h, srcp2, dstp2, zinit)


def _tc_step(h, parts, W1, b1r, W2, b2r, fp_in, blk):
    n, d = h.shape
    f = W2.shape[1]
    grid = (n // blk,)

    def body(h_ref, p_ref, w1_ref, b1_ref, w2_ref, b2_ref, fpin_ref,
             hn_ref, fp_ref):
        upd = h_ref[...] + p_ref[0] + p_ref[1]
        hsh = jnp.maximum(
            jnp.dot(upd, w1_ref[...], preferred_element_type=jnp.float32)
            + b1_ref[...], 0.0)
        hn_ref[...] = hsh
        logits = (jnp.dot(hsh.astype(jnp.bfloat16),
                          w2_ref[...].astype(jnp.bfloat16),
                          preferred_element_type=jnp.float32)
                  + b2_ref[...])
        m = jnp.max(logits, axis=1, keepdims=True)
        e = jnp.exp(logits - m)
        ssum = jnp.sum(e, axis=1, keepdims=True)
        contrib = jnp.sum(e / ssum, axis=0, keepdims=True)

        @pl.when(pl.program_id(0) == 0)
        def _init():
            fp_ref[...] = fpin_ref[...]

        fp_ref[...] += contrib

    return pl.pallas_call(
        body,
        grid=grid,
        in_specs=[
            pl.BlockSpec((blk, d), lambda i: (i, 0)),
            pl.BlockSpec((NC, blk, d), lambda i: (0, i, 0)),
            pl.BlockSpec((d, d), lambda i: (0, 0)),
            pl.BlockSpec((1, d), lambda i: (0, 0)),
            pl.BlockSpec((d, f), lambda i: (0, 0)),
            pl.BlockSpec((1, f), lambda i: (0, 0)),
            pl.BlockSpec((1, f), lambda i: (0, 0)),
        ],
        out_specs=[
            pl.BlockSpec((blk, d), lambda i: (i, 0)),
            pl.BlockSpec((1, f), lambda i: (0, 0)),
        ],
        out_shape=[
            jax.ShapeDtypeStruct((n, d), jnp.float32),
            jax.ShapeDtypeStruct((1, f), jnp.float32),
        ],
    )(h, parts, W1, b1r, W2, b2r, fp_in)


def kernel(x, edge_index, W1, b1, W2, b2):
    n, d = x.shape
    f = W2.shape[1]
    src = edge_index[0].astype(jnp.int32)
    dst = edge_index[1].astype(jnp.int32)
    e = src.shape[0]

    nw = NC * NS
    per_round = nw * CHUNK * NBUF * NHALF
    ep = ((e + per_round - 1) // per_round) * per_round
    chunks_per_worker = ep // (nw * CHUNK)
    npad = ((n + 1 + NS * 8 - 1) // (NS * 8)) * (NS * 8)
    pad_ar = jnp.arange(ep - e, dtype=jnp.int32)
    pad_src = pad_ar % n
    pad_dst = n + pad_ar % (npad - n)

    assert n <= (1 << 14)
    subg = 32
    ngrp = nw * subg
    packed = jnp.concatenate([(src << 14) + dst, (pad_src << 14) + pad_dst])
    packed = packed.reshape(ngrp, ep // ngrp)
    span = n << 14
    offs = ((jnp.arange(ngrp, dtype=jnp.int32) * (n // ngrp)) << 14)[:, None]
    packed = (jnp.sort((packed - offs) % span, axis=1) + offs) % span
    srcp_flat = packed >> 14
    dstp_flat = packed & ((1 << 14) - 1)

    def _layout(a):
        a = a.reshape(nw, chunks_per_worker, CHUNK)
        a = jnp.swapaxes(a, 1, 2)
        return a.reshape(-1, CHUNK)

    srcp = _layout(srcp_flat)
    dstp = _layout(dstp_flat)
    zinit = jnp.zeros((npad, d), jnp.float32)

    b1r = b1.reshape(1, d)
    b2r = b2.reshape(1, f)
    fp = jnp.zeros((1, f), jnp.float32)
    h = x
    for _ in range(RADIUS):
        parts = _sc_segment_sum(h, srcp, dstp, zinit, n, npad, chunks_per_worker)
        h, fp = _tc_step(h, parts, W1, b1r, W2, b2r, fp, blk=1000)
    return fp


# speedup vs baseline: 1.5801x; 1.0810x over previous
import functools

import jax
import jax.numpy as jnp
from jax import lax
from jax.experimental import pallas as pl
from jax.experimental.pallas import tpu as pltpu
from jax.experimental.pallas import tpu_sc as plsc

NC = 2
NS = 16
CHUNK = 64
NBUF = 4
NHALF = 4
RADIUS = 3


def _sc_segment_sum(h, srcp2, dstp2, zinit, n, npad, chunks_per_worker):
    d = h.shape[1]
    t_w = chunks_per_worker
    t_h = t_w // NHALF
    rows_per_tile = npad // NS
    mesh = plsc.VectorSubcoreMesh(core_axis_name="c", subcore_axis_name="s")

    @functools.partial(
        pl.kernel,
        out_type=jax.ShapeDtypeStruct((NC, npad, d), jnp.float32),
        mesh=mesh,
        scratch_types=[
            pltpu.VMEM((t_h, CHUNK), jnp.int32),
            pltpu.VMEM((t_h, CHUNK), jnp.int32),
            pltpu.VMEM((NBUF, CHUNK, d), jnp.float32),
            pltpu.VMEM_SHARED((npad, d), jnp.float32),
            pltpu.SemaphoreType.DMA((NBUF,)),
        ],
    )
    def seg_kernel(h_hbm, src_hbm, dst_hbm, z_hbm, out_hbm, sidx, didx, rows, acc, sems):
        c = lax.axis_index("c")
        s = lax.axis_index("s")
        wid = c * NS + s
        cbase = wid * t_w
        pltpu.sync_copy(z_hbm.at[pl.ds(s * rows_per_tile, rows_per_tile)],
                        acc.at[pl.ds(s * rows_per_tile, rows_per_tile)])
        plsc.subcore_barrier()

        def gather_start(t, b):
            pltpu.async_copy(h_hbm.at[sidx.at[t]], rows.at[b], sems.at[b])

        def gather_wait(t, b):
            pltpu.make_async_copy(h_hbm.at[sidx.at[t]], rows.at[b], sems.at[b]).wait()

        for half in range(NHALF):
            pltpu.sync_copy(src_hbm.at[pl.ds(cbase + half * t_h, t_h)], sidx)
            pltpu.sync_copy(dst_hbm.at[pl.ds(cbase + half * t_h, t_h)], didx)
            for b in range(NBUF):
                gather_start(b, b)

            def outer(i, carry):
                t0 = i * NBUF
                for b in range(NBUF):
                    t = t0 + b
                    gather_wait(t, b)
                    pltpu.sync_copy(rows.at[b], acc.at[didx.at[t]], add=True)

                    @pl.when(t + NBUF < t_h)
                    def _next():
                        gather_start(t + NBUF, b)
                return carry

            lax.fori_loop(0, t_h // NBUF, outer, 0, unroll=False)
        plsc.subcore_barrier()
        pltpu.sync_copy(acc.at[pl.ds(s * rows_per_tile, rows_per_tile)],
                        out_hbm.at[c, pl.ds(s * rows_per_tile, rows_per_tile)])

    return seg_kernel(h, srcp2, dstp2, zinit)


def _tc_hash(h, parts, W1, b1r, blk):
    n, d = h.shape
    grid = (n // blk,)

    def body(h_ref, p_ref, w1_ref, b1_ref, hn_ref):
        upd = h_ref[...] + p_ref[0] + p_ref[1]
        hn_ref[...] = jnp.maximum(
            jnp.dot(upd, w1_ref[...], preferred_element_type=jnp.float32)
            + b1_ref[...], 0.0)

    return pl.pallas_call(
        body,
        grid=grid,
        in_specs=[
            pl.BlockSpec((blk, d), lambda i: (i, 0)),
            pl.BlockSpec((NC, blk, d), lambda i: (0, i, 0)),
            pl.BlockSpec((d, d), lambda i: (0, 0)),
            pl.BlockSpec((1, d), lambda i: (0, 0)),
        ],
        out_specs=pl.BlockSpec((blk, d), lambda i: (i, 0)),
        out_shape=jax.ShapeDtypeStruct((n, d), jnp.float32),
    )(h, parts, W1, b1r)


def _tc_fp(hsh, W2, b2r, fp_in, blk):
    n, d = hsh.shape
    f = W2.shape[1]
    grid = (n // blk,)

    def body(hs_ref, w2_ref, b2_ref, fpin_ref, fp_ref):
        logits = (jnp.dot(hs_ref[...], w2_ref[...],
                          preferred_element_type=jnp.float32) + b2_ref[...])
        m = jnp.max(logits, axis=1, keepdims=True)
        e = jnp.exp(logits - m)
        ssum = jnp.sum(e, axis=1, keepdims=True)
        contrib = jnp.sum(e / ssum, axis=0, keepdims=True)

        @pl.when(pl.program_id(0) == 0)
        def _init():
            fp_ref[...] = fpin_ref[...]

        fp_ref[...] += contrib

    return pl.pallas_call(
        body,
        grid=grid,
        in_specs=[
            pl.BlockSpec((blk, d), lambda i: (i, 0)),
            pl.BlockSpec((d, f), lambda i: (0, 0)),
            pl.BlockSpec((1, f), lambda i: (0, 0)),
            pl.BlockSpec((1, f), lambda i: (0, 0)),
        ],
        out_specs=pl.BlockSpec((1, f), lambda i: (0, 0)),
        out_shape=jax.ShapeDtypeStruct((1, f), jnp.float32),
    )(hsh, W2, b2r, fp_in)


def kernel(x, edge_index, W1, b1, W2, b2):
    n, d = x.shape
    f = W2.shape[1]
    src = edge_index[0].astype(jnp.int32)
    dst = edge_index[1].astype(jnp.int32)
    e = src.shape[0]

    nw = NC * NS
    per_round = nw * CHUNK * NBUF * NHALF
    ep = ((e + per_round - 1) // per_round) * per_round
    chunks_per_worker = ep // (nw * CHUNK)
    npad = ((n + 1 + NS * 8 - 1) // (NS * 8)) * (NS * 8)
    pad_ar = jnp.arange(ep - e, dtype=jnp.int32)
    pad_src = pad_ar % n
    pad_dst = n + pad_ar % (npad - n)

    assert n <= (1 << 14)
    subg = 32
    ngrp = nw * subg
    packed = jnp.concatenate([(src << 14) + dst, (pad_src << 14) + pad_dst])
    packed = packed.reshape(ngrp, ep // ngrp)
    span = n << 14
    offs = ((jnp.arange(ngrp, dtype=jnp.int32) * (n // ngrp)) << 14)[:, None]
    packed = (jnp.sort((packed - offs) % span, axis=1) + offs) % span
    srcp_flat = packed >> 14
    dstp_flat = packed & ((1 << 14) - 1)

    def _layout(a):
        a = a.reshape(nw, chunks_per_worker, CHUNK)
        a = jnp.swapaxes(a, 1, 2)
        return a.reshape(-1, CHUNK)

    srcp = _layout(srcp_flat)
    dstp = _layout(dstp_flat)
    zinit = jnp.zeros((npad, d), jnp.float32)

    b1r = b1.reshape(1, d)
    b2r = b2.reshape(1, f)
    fp = jnp.zeros((1, f), jnp.float32)
    h = x
    for _ in range(RADIUS):
        parts = _sc_segment_sum(h, srcp, dstp, zinit, n, npad, chunks_per_worker)
        h = _tc_hash(h, parts, W1, b1r, blk=1000)
        fp = _tc_fp(h, W2, b2r, fp, blk=1000)
    return fp
